# baseline (speedup 1.0000x reference)
import functools

import jax
import jax.numpy as jnp
from jax import lax
from jax.experimental import pallas as pl
from jax.experimental.pallas import tpu as pltpu
from jax.experimental.pallas import tpu_sc as plsc

N = 10000
E = 160000
NG = 256
NF = 39
EF = 11
G = 200

NP = 10112
H = 128
LO = H - 1
SCOL = G - LO
PCOL_HI = SCOL + 1
NCOL_HI = SCOL + 2
XW = 128
EAW = 16
F32 = jnp.float32
BF16 = jnp.bfloat16
I32 = jnp.int32

_dot = functools.partial(jnp.dot, preferred_element_type=jnp.float32)


def _lrelu(v):
    return jnp.maximum(v, 0.01 * v)


def _elu(v):
    return jnp.where(v > 0.0, v, jnp.exp(jnp.minimum(v, 0.0)) - 1.0)


def _gru(xv, hv, wr, wz, wn, ur, uz, un, br, bz, bn_, cr, cz, cn):
    r = jax.nn.sigmoid(_dot(xv, wr[...]) + br[...] + _dot(hv, ur[...]) + cr[...])
    z = jax.nn.sigmoid(_dot(xv, wz[...]) + bz[...] + _dot(hv, uz[...]) + cz[...])
    n = jnp.tanh(_dot(xv, wn[...]) + bn_[...] + r * (_dot(hv, un[...]) + cn[...]))
    return (1.0 - z) * n + z * hv


def _gru_split(Wih, Whh, bih, bhh):
    return (Wih[:, :G], Wih[:, G:2 * G], Wih[:, 2 * G:],
            Whh[:, :G], Whh[:, G:2 * G], Whh[:, 2 * G:],
            bih[None, :G], bih[None, G:2 * G], bih[None, 2 * G:],
            bhh[None, :G], bhh[None, G:2 * G], bhh[None, 2 * G:])


def _sc_gather_x(xpad, src):
    mesh = plsc.VectorSubcoreMesh(core_axis_name="c", subcore_axis_name="s")
    NW = 32
    EW = E // NW
    CH = 128
    NCH = (EW + CH - 1) // CH

    NB = 4

    @functools.partial(
        pl.kernel,
        out_type=jax.ShapeDtypeStruct((E, XW), F32),
        mesh=mesh,
        compiler_params=pltpu.CompilerParams(needs_layout_passes=False),
        scratch_types=[
            [pltpu.VMEM((CH,), I32)] * NB,
            [pltpu.VMEM((CH, XW), F32)] * NB,
            [pltpu.SemaphoreType.DMA] * NB,
            [pltpu.SemaphoreType.DMA] * NB,
            [pltpu.SemaphoreType.DMA] * NB,
        ],
    )
    def k(x_hbm, src_hbm, out_hbm, idx_v, rows_v, isem, gsem, osem):
        c = lax.axis_index("c")
        s = lax.axis_index("s")
        wid = s * 2 + c
        base0 = wid * EW

        def boff(j):
            return base0 + jnp.minimum(j * CH, EW - CH)

        def issue_idx(j, t):
            pltpu.async_copy(src_hbm.at[pl.ds(boff(j), CH)], idx_v[t], isem[t])

        def slot(k_, t0):
            t2 = (t0 + 2) % NB
            t1 = (t0 + 1) % NB
            j2 = k_ + 2
            j1 = k_ + 1

            @pl.when(j2 < NCH)
            def _():
                issue_idx(j2, t2)

            @pl.when(j1 < NCH)
            def _():
                pltpu.make_async_copy(
                    src_hbm.at[pl.ds(boff(j1), CH)], idx_v[t1], isem[t1]).wait()

                @pl.when(j1 >= NB)
                def _():
                    pltpu.make_async_copy(
                        rows_v[t1], out_hbm.at[pl.ds(boff(j1 - NB), CH)],
                        osem[t1]).wait()

                pltpu.async_copy(x_hbm.at[idx_v[t1]], rows_v[t1], gsem[t1])

            pltpu.make_async_copy(x_hbm.at[idx_v[t0]], rows_v[t0],
                                  gsem[t0]).wait()
            pltpu.async_copy(rows_v[t0], out_hbm.at[pl.ds(boff(k_), CH)],
                             osem[t0])

        issue_idx(0, 0)
        pltpu.make_async_copy(src_hbm.at[pl.ds(boff(0), CH)], idx_v[0],
                              isem[0]).wait()
        pltpu.async_copy(x_hbm.at[idx_v[0]], rows_v[0], gsem[0])
        issue_idx(1, 1)

        def big(m, _):
            for t in range(NB):
                j = m * NB + t

                @pl.when(j < NCH)
                def _():
                    slot(j, t)

            return 0

        lax.fori_loop(0, (NCH + NB - 1) // NB, big, 0)
        for j in range(NCH - NB, NCH):
            pltpu.make_async_copy(rows_v[j % NB],
                                  out_hbm.at[pl.ds(boff(j), CH)],
                                  osem[j % NB]).wait()

    return k(xpad, src)


def _sc_scatter1(lo, hi, q1, dst, zrows):
    mesh = plsc.VectorSubcoreMesh(core_axis_name="c", subcore_axis_name="s")
    EPT = E // 16
    CH = 80
    NCH = EPT // CH
    NPT = NP // 16

    @functools.partial(
        pl.kernel,
        out_type=(jax.ShapeDtypeStruct((NP, H), F32),
                  jax.ShapeDtypeStruct((NP, H), F32)),
        mesh=mesh,
        compiler_params=pltpu.CompilerParams(needs_layout_passes=False),
        scratch_types=[
            pltpu.VMEM((NP,), F32),
            [pltpu.VMEM((CH,), I32)] * 3,
            pltpu.VMEM((CH,), F32),
            [pltpu.VMEM((CH, H), F32)] * 3,
            pltpu.VMEM_SHARED((NP, H), F32),
            [pltpu.SemaphoreType.DMA] * 3,
            [pltpu.SemaphoreType.DMA] * 3,
        ],
    )
    def k(lo_hbm, hi_hbm, q_hbm, dst_hbm, z_hbm, olo, ohi,
          qtab, dstb, wb, rows, acc, lsem, ssem):
        c = lax.axis_index("c")
        s = lax.axis_index("s")
        pltpu.sync_copy(q_hbm, qtab)
        pltpu.sync_copy(z_hbm, acc.at[pl.ds(s * NPT, NPT)])
        plsc.subcore_barrier()
        NB = 3

        def run(rows_hbm, ncols16, pcol):
            e0 = s * EPT
            lanes = lax.iota(I32, 16)

            def issue_load(j, t):
                b = e0 + j * CH
                pltpu.async_copy(dst_hbm.at[pl.ds(b, CH)], dstb[t], lsem[t])
                pltpu.async_copy(rows_hbm.at[pl.ds(b, CH)], rows[t], lsem[t])

            def wait_load(j, t):
                b = e0 + j * CH
                pltpu.make_async_copy(dst_hbm.at[pl.ds(b, CH)], dstb[t],
                                      lsem[t]).wait()
                pltpu.make_async_copy(rows_hbm.at[pl.ds(b, CH)], rows[t],
                                      lsem[t]).wait()

            def wait_scat(t):
                pltpu.make_async_copy(rows[t], acc.at[dstb[t]],
                                      ssem[t]).wait()

            def scale(t):
                def grpw(g, _):
                    sl = pl.ds(g * 16, 16)
                    e16 = g * 16 + lanes
                    pv = plsc.load_gather(rows[t], [e16, jnp.zeros((16,), I32)
                                                    + pcol])
                    tv = plsc.load_gather(qtab, [dstb[t][sl]]) + pv
                    wb[sl] = jnp.exp(jnp.maximum(tv, 0.01 * tv))
                    return 0

                lax.fori_loop(0, CH // 16, grpw, 0)

                def edge(e, _):
                    wsp = plsc.load_gather(wb, [jnp.zeros((16,), I32) + e])
                    for kk in range(ncols16):
                        cs = pl.ds(kk * 16, 16)
                        rows[t][e, cs] = rows[t][e, cs] * wsp
                    return 0

                lax.fori_loop(0, CH, edge, 0, unroll=8)

            def slot(j, t0):
                t1 = (t0 + 1) % NB
                j1 = j + 1

                @pl.when(j1 < NCH)
                def _():
                    @pl.when(j1 >= NB)
                    def _():
                        wait_scat(t1)

                    issue_load(j1, t1)

                wait_load(j, t0)
                scale(t0)
                pltpu.async_copy(rows[t0], acc.at[dstb[t0]], ssem[t0],
                                 add=True)

            issue_load(0, 0)

            def big(m, _):
                for t in range(NB):
                    j = m * NB + t

                    @pl.when(j < NCH)
                    def _():
                        slot(j, t)

                return 0

            lax.fori_loop(0, (NCH + NB - 1) // NB, big, 0)
            for j in range(NCH - NB, NCH):
                wait_scat(j % NB)

        @pl.when(c == 0)
        def _():
            run(lo_hbm, H // 16, LO)

        @pl.when(c == 1)
        def _():
            run(hi_hbm, (NCOL_HI + 15) // 16, PCOL_HI)

        plsc.subcore_barrier()
        r0 = s * NPT

        @pl.when(c == 0)
        def _():
            pltpu.sync_copy(acc.at[pl.ds(r0, NPT)], olo.at[pl.ds(r0, NPT)])

        @pl.when(c == 1)
        def _():
            pltpu.sync_copy(acc.at[pl.ds(r0, NPT)], ohi.at[pl.ds(r0, NPT)])

    return k(lo, hi, q1, dst, zrows)


def _sc_scatter2(pn_lo, pn_hi, qd, src, dst, zrows):
    mesh = plsc.VectorSubcoreMesh(core_axis_name="c", subcore_axis_name="s")
    EPT = E // 16
    CH = 80
    NCH = EPT // CH
    NPT = NP // 16

    @functools.partial(
        pl.kernel,
        out_type=(jax.ShapeDtypeStruct((NP, H), F32),
                  jax.ShapeDtypeStruct((NP, H), F32)),
        mesh=mesh,
        compiler_params=pltpu.CompilerParams(needs_layout_passes=False),
        scratch_types=[
            pltpu.VMEM((NP,), F32),
            [pltpu.VMEM((CH,), I32)] * 3,
            [pltpu.VMEM((CH,), I32)] * 3,
            pltpu.VMEM((CH,), F32),
            [pltpu.VMEM((CH, H), F32)] * 2,
            pltpu.VMEM_SHARED((NP, H), F32),
            [pltpu.SemaphoreType.DMA] * 3,
            [pltpu.SemaphoreType.DMA] * 2,
            [pltpu.SemaphoreType.DMA] * 2,
        ],
    )
    def k(lo_hbm, hi_hbm, qd_hbm, src_hbm, dst_hbm, z_hbm, olo, ohi,
          qdtab, dstb, srcb, wb, rows, acc, lsem, gsem, ssem):
        c = lax.axis_index("c")
        s = lax.axis_index("s")
        pltpu.sync_copy(qd_hbm, qdtab)
        pltpu.sync_copy(z_hbm, acc.at[pl.ds(s * NPT, NPT)])
        plsc.subcore_barrier()

        def run(rows_hbm, ncols16, pcol):
            e0 = s * EPT
            lanes = lax.iota(I32, 16)

            def issue_idx(j, t3):
                b = e0 + j * CH
                pltpu.async_copy(dst_hbm.at[pl.ds(b, CH)], dstb[t3], lsem[t3])
                pltpu.async_copy(src_hbm.at[pl.ds(b, CH)], srcb[t3], lsem[t3])

            def wait_idx(j, t3):
                b = e0 + j * CH
                pltpu.make_async_copy(dst_hbm.at[pl.ds(b, CH)], dstb[t3],
                                      lsem[t3]).wait()
                pltpu.make_async_copy(src_hbm.at[pl.ds(b, CH)], srcb[t3],
                                      lsem[t3]).wait()

            def wait_scat(t2, t3):
                pltpu.make_async_copy(rows[t2], acc.at[dstb[t3]],
                                      ssem[t2]).wait()

            def scale(t2, t3):
                def grpw(g, _):
                    sl = pl.ds(g * 16, 16)
                    e16 = g * 16 + lanes
                    qsv = plsc.load_gather(rows[t2], [e16, jnp.zeros((16,), I32)
                                                      + pcol])
                    tv = plsc.load_gather(qdtab, [dstb[t3][sl]]) + qsv
                    wb[sl] = jnp.exp(jnp.maximum(tv, 0.01 * tv))
                    return 0

                lax.fori_loop(0, CH // 16, grpw, 0)

                def edge(e, _):
                    wsp = plsc.load_gather(wb, [jnp.zeros((16,), I32) + e])
                    for kk in range(ncols16):
                        cs = pl.ds(kk * 16, 16)
                        rows[t2][e, cs] = rows[t2][e, cs] * wsp
                    return 0

                lax.fori_loop(0, CH, edge, 0, unroll=8)

            def slot(j, t):
                t2 = t % 2
                t3 = t % 3
                j1 = j + 1
                j2 = j + 2

                def drain():
                    wait_scat((t - 1) % 2, (t - 1) % 3)

                if t == 0:
                    @pl.when(j >= 1)
                    def _():
                        drain()
                else:
                    drain()

                @pl.when(j2 < NCH)
                def _():
                    issue_idx(j2, (t + 2) % 3)

                @pl.when(j1 < NCH)
                def _():
                    wait_idx(j1, (t + 1) % 3)
                    pltpu.async_copy(rows_hbm.at[srcb[(t + 1) % 3]],
                                     rows[(t + 1) % 2], gsem[(t + 1) % 2])

                pltpu.make_async_copy(rows_hbm.at[srcb[t3]], rows[t2],
                                      gsem[t2]).wait()
                scale(t2, t3)
                pltpu.async_copy(rows[t2], acc.at[dstb[t3]], ssem[t2],
                                 add=True)

            issue_idx(0, 0)
            issue_idx(1, 1)
            wait_idx(0, 0)
            pltpu.async_copy(rows_hbm.at[srcb[0]], rows[0], gsem[0])

            def big(m, _):
                for t in range(6):
                    j = m * 6 + t

                    @pl.when(j < NCH)
                    def _():
                        slot(j, t)

                return 0

            lax.fori_loop(0, (NCH + 5) // 6, big, 0)
            wait_scat((NCH - 1) % 2, (NCH - 1) % 3)

        @pl.when(c == 0)
        def _():
            run(lo_hbm, H // 16, LO)

        @pl.when(c == 1)
        def _():
            run(hi_hbm, (NCOL_HI + 15) // 16, PCOL_HI)

        plsc.subcore_barrier()
        r0 = s * NPT

        @pl.when(c == 0)
        def _():
            pltpu.sync_copy(acc.at[pl.ds(r0, NPT)], olo.at[pl.ds(r0, NPT)])

        @pl.when(c == 1)
        def _():
            pltpu.sync_copy(acc.at[pl.ds(r0, NPT)], ohi.at[pl.ds(r0, NPT)])

    return k(pn_lo, pn_hi, qd, src, dst, zrows)


def _tc_node1(xpad, Wpn, bpn, Wq, bq):
    BN = 1264

    def body(x_ref, wpn, bpn_r, wq, bq_r, hv_ref, q_ref):
        hv = _lrelu(_dot(x_ref[...], wpn[...]) + bpn_r[...])
        hv_ref[...] = hv
        q_ref[...] = _dot(hv, wq[...]) + bq_r[...]

    return pl.pallas_call(
        body,
        grid=(NP // BN,),
        in_specs=[
            pl.BlockSpec((BN, XW), lambda i: (i, 0)),
            pl.BlockSpec((XW, G), lambda i: (0, 0)),
            pl.BlockSpec((1, G), lambda i: (0, 0)),
            pl.BlockSpec((G, 1), lambda i: (0, 0)),
            pl.BlockSpec((1, 1), lambda i: (0, 0)),
        ],
        out_specs=[
            pl.BlockSpec((BN, G), lambda i: (i, 0)),
            pl.BlockSpec((BN, 1), lambda i: (i, 0)),
        ],
        out_shape=[
            jax.ShapeDtypeStruct((NP, G), F32),
            jax.ShapeDtypeStruct((NP, 1), F32),
        ],
    )(xpad, Wpn, bpn, Wq, bq)


def _tc_edge(xg, ea, Wa, Wb, b1, Wp):
    BE = 2000

    def body(xg_ref, ea_ref, wa, wb_r, b1_r, wp, lo_ref, hi_ref):
        he = _lrelu(_dot(xg_ref[...], wa[...])
                    + _dot(ea_ref[...], wb_r[...]) + b1_r[...])
        pcol = _dot(he, wp[...])
        lo_ref[...] = jnp.concatenate([he[:, :LO], pcol], axis=1)
        hi_ref[...] = jnp.concatenate(
            [he[:, LO:], jnp.ones((BE, 1), F32), pcol,
             jnp.zeros((BE, H - NCOL_HI), F32)], axis=1)

    return pl.pallas_call(
        body,
        grid=(E // BE,),
        in_specs=[
            pl.BlockSpec((BE, XW), lambda i: (i, 0)),
            pl.BlockSpec((BE, EF), lambda i: (i, 0)),
            pl.BlockSpec((XW, G), lambda i: (0, 0)),
            pl.BlockSpec((EF, G), lambda i: (0, 0)),
            pl.BlockSpec((1, G), lambda i: (0, 0)),
            pl.BlockSpec((G, 1), lambda i: (0, 0)),
        ],
        out_specs=[
            pl.BlockSpec((BE, H), lambda i: (i, 0)),
            pl.BlockSpec((BE, H), lambda i: (i, 0)),
        ],
        out_shape=[
            jax.ShapeDtypeStruct((E, H), F32),
            jax.ShapeDtypeStruct((E, H), F32),
        ],
    )(xg, ea, Wa, Wb, b1, Wp)


def _tc_node2(acc_lo, acc_hi, hv, Wet, bet, gru1, Wplt, Wplb, bpl, Wpn2, bpn2):
    BN = 1264

    def body(alo, ahi, hv_ref, wet, bet_r, *rest):
        g1 = rest[:12]
        wplt, wplb, bpl_r, wpn2, bpn2_r = rest[12:17]
        h_o, qd_o, plo_o, phi_o = rest[17:]
        ahiv = ahi[...]
        sden = ahiv[:, SCOL:SCOL + 1]
        safe = jnp.where(sden == 0.0, 1.0, sden)
        ind = (sden > 0.0).astype(F32)
        accf = jnp.concatenate([alo[:, :LO], ahiv[:, :SCOL]], axis=1) / safe
        ctx = _elu(_dot(accf, wet[...]) + bet_r[...] * ind)
        h = jax.nn.relu(_gru(ctx, hv_ref[...], *g1))
        h_o[...] = h
        qd_o[...] = _dot(h, wplt[...])
        qs = _dot(h, wplb[...]) + bpl_r[...]
        pn = _dot(h, wpn2[...]) + bpn2_r[...]
        plo_o[...] = jnp.concatenate([pn[:, :LO], qs], axis=1)
        phi_o[...] = jnp.concatenate(
            [pn[:, LO:], jnp.ones((BN, 1), F32), qs,
             jnp.zeros((BN, H - NCOL_HI), F32)], axis=1)

    full = lambda shape: pl.BlockSpec(shape, lambda i: tuple(0 for _ in shape))
    return pl.pallas_call(
        body,
        grid=(NP // BN,),
        in_specs=[
            pl.BlockSpec((BN, H), lambda i: (i, 0)),
            pl.BlockSpec((BN, H), lambda i: (i, 0)),
            pl.BlockSpec((BN, G), lambda i: (i, 0)),
            full((G, G)), full((1, G)),
            *[full(w.shape) for w in gru1],
            full((G, 1)), full((G, 1)), full((1, 1)),
            full((G, G)), full((1, G)),
        ],
        out_specs=[
            pl.BlockSpec((BN, G), lambda i: (i, 0)),
            pl.BlockSpec((BN, 1), lambda i: (i, 0)),
            pl.BlockSpec((BN, H), lambda i: (i, 0)),
            pl.BlockSpec((BN, H), lambda i: (i, 0)),
        ],
        out_shape=[
            jax.ShapeDtypeStruct((NP, G), F32),
            jax.ShapeDtypeStruct((NP, 1), F32),
            jax.ShapeDtypeStruct((NP, H), F32),
            jax.ShapeDtypeStruct((NP, H), F32),
        ],
    )(acc_lo, acc_hi, hv, Wet, bet, *gru1, Wplt, Wplb, bpl, Wpn2, bpn2)


def _tc_readout(acc_lo, acc_hi, h, gid_row, gid_col, plist):
    CK = 632
    NCH = NP // CK
    npar = len(plist)

    def body(*refs):
        alo, ahi, h_ref, gr, gc = refs[:5]
        prefs = refs[5:5 + npar]
        out_ref = refs[5 + npar]
        h2s = refs[5 + npar + 1]
        g2w = prefs[0:12]
        i256c = lax.broadcasted_iota(I32, (NG, 1), 0)
        i256r = lax.broadcasted_iota(I32, (1, NG), 1)

        def passA(i, g0):
            slr = pl.ds(i * CK, CK)
            ahic = ahi[slr, :]
            s2 = ahic[:, SCOL:SCOL + 1]
            safe2 = jnp.where(s2 == 0.0, 1.0, s2)
            accf = jnp.concatenate([alo[slr, :LO], ahic[:, :SCOL]],
                                   axis=1) / safe2
            ctx2 = _elu(accf)
            h2c = jax.nn.relu(_gru(ctx2, h_ref[slr, :], *g2w))
            h2s[slr, :] = h2c
            S = (i256c == gr[pl.ds(i, 1), :]).astype(F32)
            return g0 + _dot(S, h2c)

        g = lax.fori_loop(0, NCH, passA, jnp.zeros((NG, G), F32))

        idx = 12
        for _t in range(2):
            wclt, wclb, bcl, wprn, bprn = prefs[idx:idx + 5]
            grt = prefs[idx + 5:idx + 17]
            idx += 17
            qg = _dot(jax.nn.relu(g), wclt[...])

            def passB(i, carry):
                acc, ss = carry
                slr = pl.ds(i * CK, CK)
                h2c = h2s[slr, :]
                qh = _dot(h2c, wclb[...]) + bcl[...]
                St = (gc[slr, :] == i256r).astype(F32)
                tt = _dot(St, qg) + qh
                w = jnp.exp(jnp.maximum(tt, 0.01 * tt))
                S = (i256c == gr[pl.ds(i, 1), :]).astype(F32)
                return acc + _dot(S, h2c * w), ss + _dot(S, w)

            acc, ss = lax.fori_loop(
                0, NCH, passB,
                (jnp.zeros((NG, G), F32), jnp.zeros((NG, 1), F32)))
            safe = jnp.where(ss == 0.0, 1.0, ss)
            ind = (ss > 0.0).astype(F32)
            ctx = _elu(_dot(acc / safe, wprn[...]) + bprn[...] * ind)
            g = jax.nn.relu(_gru(ctx, g, *grt))

        out_ref[...] = _dot(g, prefs[-2][...]) + prefs[-1][...]

    vspec = pl.BlockSpec(memory_space=pltpu.VMEM)
    return pl.pallas_call(
        body,
        in_specs=[vspec] * (5 + npar),
        out_specs=vspec,
        out_shape=jax.ShapeDtypeStruct((NG, 1), F32),
        scratch_shapes=[pltpu.VMEM((NP, G), F32)],
    )(acc_lo, acc_hi, h, gid_row, gid_col, *plist)


def kernel(x, edge_attr, edge_index, node_graph_ids, params):
    p = params
    src = edge_index[0]
    dst = edge_index[1]

    xpad = jnp.pad(x, ((0, NP - N), (0, XW - NF)))
    Wa = jnp.pad(p['W_pe1'][:NF], ((0, XW - NF), (0, 0)))
    Wb = p['W_pe1'][NF:]
    Wpn = jnp.pad(p['W_pn'], ((0, XW - NF), (0, 0)))
    bpn = p['b_pn'][None, :]
    Wq1 = p['W_pe2'][:G]
    bq1 = p['b_pe2'][None, :]
    Wp1 = p['W_pe2'][G:]
    b_pe1 = p['b_pe1'][None, :]
    gru1 = _gru_split(p['Wih1'], p['Whh1'], p['bih1'], p['bhh1'])
    gru2 = _gru_split(p['Wih2'], p['Whh2'], p['bih2'], p['bhh2'])
    Wplt = p['W_pe_l'][:G]
    Wplb = p['W_pe_l'][G:]
    bpl = p['b_pe_l'][None, :]
    zrows = jnp.zeros((NP // 16, H), F32)
    gid_pad = jnp.pad(node_graph_ids, (0, NP - N), constant_values=NG + 7)
    gid_row = gid_pad.reshape(NP // 632, 632)
    gid_col = gid_pad[:, None]

    plist = list(gru2)
    for t in range(2):
        plist += [p['Wcl%d' % t][:G], p['Wcl%d' % t][G:], p['bcl%d' % t][None, :],
                  p['Wprn%d' % t], p['bprn%d' % t][None, :]]
        plist += list(_gru_split(p['Wih_r%d' % t], p['Whh_r%d' % t],
                                 p['bih_r%d' % t], p['bhh_r%d' % t]))
    plist += [p['W_out'], p['b_out'][None, :]]

    xg = _sc_gather_x(xpad, src)
    hv, q1c = _tc_node1(xpad, Wpn, bpn, Wq1, bq1)
    lo1, hi1 = _tc_edge(xg, edge_attr, Wa, Wb, b_pe1, Wp1)
    acc_lo, acc_hi = _sc_scatter1(lo1, hi1, q1c.reshape(NP), dst, zrows)
    h, qd_c, pn_lo, pn_hi = _tc_node2(acc_lo, acc_hi, hv, p['W_et'],
                                      p['b_et'][None, :], gru1,
                                      Wplt, Wplb, bpl,
                                      p['W_pn2'], p['b_pn2'][None, :])
    acc2_lo, acc2_hi = _sc_scatter2(pn_lo, pn_hi, qd_c.reshape(NP), src, dst,
                                    zrows)
    return _tc_readout(acc2_lo, acc2_hi, h, gid_row, gid_col, plist)


# speedup vs baseline: 1.0173x; 1.0173x over previous
import functools

import jax
import jax.numpy as jnp
from jax import lax
from jax.experimental import pallas as pl
from jax.experimental.pallas import tpu as pltpu
from jax.experimental.pallas import tpu_sc as plsc

N = 10000
E = 160000
NG = 256
NF = 39
EF = 11
G = 200

NP = 10112
H = 128
LO = H - 1
SCOL = G - LO
PCOL_HI = SCOL + 1
NCOL_HI = SCOL + 2
XW = 128
EAW = 16
F32 = jnp.float32
BF16 = jnp.bfloat16
I32 = jnp.int32

_dot = functools.partial(jnp.dot, preferred_element_type=jnp.float32)


def _lrelu(v):
    return jnp.maximum(v, 0.01 * v)


def _elu(v):
    return jnp.where(v > 0.0, v, jnp.exp(jnp.minimum(v, 0.0)) - 1.0)


def _gru(xv, hv, wr, wz, wn, ur, uz, un, br, bz, bn_, cr, cz, cn):
    r = jax.nn.sigmoid(_dot(xv, wr[...]) + br[...] + _dot(hv, ur[...]) + cr[...])
    z = jax.nn.sigmoid(_dot(xv, wz[...]) + bz[...] + _dot(hv, uz[...]) + cz[...])
    n = jnp.tanh(_dot(xv, wn[...]) + bn_[...] + r * (_dot(hv, un[...]) + cn[...]))
    return (1.0 - z) * n + z * hv


def _gru_split(Wih, Whh, bih, bhh):
    return (Wih[:, :G], Wih[:, G:2 * G], Wih[:, 2 * G:],
            Whh[:, :G], Whh[:, G:2 * G], Whh[:, 2 * G:],
            bih[None, :G], bih[None, G:2 * G], bih[None, 2 * G:],
            bhh[None, :G], bhh[None, G:2 * G], bhh[None, 2 * G:])


def _sc_gather_x(xpad, src):
    mesh = plsc.VectorSubcoreMesh(core_axis_name="c", subcore_axis_name="s")
    NW = 32
    EW = E // NW
    CH = 128
    NCH = (EW + CH - 1) // CH

    NB = 3

    @functools.partial(
        pl.kernel,
        out_type=jax.ShapeDtypeStruct((E, XW), F32),
        mesh=mesh,
        compiler_params=pltpu.CompilerParams(needs_layout_passes=False),
        scratch_types=[
            [pltpu.VMEM((CH,), I32)] * NB,
            [pltpu.VMEM((CH, XW), F32)] * NB,
            [pltpu.SemaphoreType.DMA] * NB,
            [pltpu.SemaphoreType.DMA] * NB,
            [pltpu.SemaphoreType.DMA] * NB,
        ],
    )
    def k(x_hbm, src_hbm, out_hbm, idx_v, rows_v, isem, gsem, osem):
        c = lax.axis_index("c")
        s = lax.axis_index("s")
        wid = s * 2 + c
        base0 = wid * EW

        def boff(j):
            return base0 + jnp.minimum(j * CH, EW - CH)

        def issue_idx(j, t):
            pltpu.async_copy(src_hbm.at[pl.ds(boff(j), CH)], idx_v[t], isem[t])

        def slot(k_, t0):
            t2 = (t0 + 2) % NB
            t1 = (t0 + 1) % NB
            j2 = k_ + 2
            j1 = k_ + 1

            @pl.when(j2 < NCH)
            def _():
                issue_idx(j2, t2)

            @pl.when(j1 < NCH)
            def _():
                pltpu.make_async_copy(
                    src_hbm.at[pl.ds(boff(j1), CH)], idx_v[t1], isem[t1]).wait()

                @pl.when(j1 >= NB)
                def _():
                    pltpu.make_async_copy(
                        rows_v[t1], out_hbm.at[pl.ds(boff(j1 - NB), CH)],
                        osem[t1]).wait()

                pltpu.async_copy(x_hbm.at[idx_v[t1]], rows_v[t1], gsem[t1])

            pltpu.make_async_copy(x_hbm.at[idx_v[t0]], rows_v[t0],
                                  gsem[t0]).wait()
            pltpu.async_copy(rows_v[t0], out_hbm.at[pl.ds(boff(k_), CH)],
                             osem[t0])

        issue_idx(0, 0)
        pltpu.make_async_copy(src_hbm.at[pl.ds(boff(0), CH)], idx_v[0],
                              isem[0]).wait()
        pltpu.async_copy(x_hbm.at[idx_v[0]], rows_v[0], gsem[0])
        issue_idx(1, 1)

        def big(m, _):
            for t in range(NB):
                j = m * NB + t

                @pl.when(j < NCH)
                def _():
                    slot(j, t)

            return 0

        lax.fori_loop(0, (NCH + NB - 1) // NB, big, 0)
        for j in range(NCH - NB, NCH):
            pltpu.make_async_copy(rows_v[j % NB],
                                  out_hbm.at[pl.ds(boff(j), CH)],
                                  osem[j % NB]).wait()

    return k(xpad, src)


def _sc_scatter1(lo, hi, q1, dst, zrows, e_lo, ne):
    mesh = plsc.VectorSubcoreMesh(core_axis_name="c", subcore_axis_name="s")
    EPT = ne // 16
    CH = 80
    NCH = EPT // CH
    NPT = NP // 16

    @functools.partial(
        pl.kernel,
        out_type=(jax.ShapeDtypeStruct((NP, H), F32),
                  jax.ShapeDtypeStruct((NP, H), F32)),
        mesh=mesh,
        compiler_params=pltpu.CompilerParams(needs_layout_passes=False),
        scratch_types=[
            pltpu.VMEM((NP,), F32),
            [pltpu.VMEM((CH,), I32)] * 3,
            pltpu.VMEM((CH,), F32),
            [pltpu.VMEM((CH, H), F32)] * 3,
            pltpu.VMEM_SHARED((NP, H), F32),
            [pltpu.SemaphoreType.DMA] * 3,
            [pltpu.SemaphoreType.DMA] * 3,
        ],
    )
    def k(lo_hbm, hi_hbm, q_hbm, dst_hbm, z_hbm, olo, ohi,
          qtab, dstb, wb, rows, acc, lsem, ssem):
        c = lax.axis_index("c")
        s = lax.axis_index("s")
        pltpu.sync_copy(q_hbm, qtab)
        pltpu.sync_copy(z_hbm, acc.at[pl.ds(s * NPT, NPT)])
        plsc.subcore_barrier()
        NB = 3

        def run(rows_hbm, ncols16, pcol):
            e0 = s * EPT
            lanes = lax.iota(I32, 16)

            def issue_load(j, t):
                b = e0 + j * CH
                pltpu.async_copy(dst_hbm.at[pl.ds(e_lo + b, CH)], dstb[t],
                                 lsem[t])
                pltpu.async_copy(rows_hbm.at[pl.ds(b, CH)], rows[t], lsem[t])

            def wait_load(j, t):
                b = e0 + j * CH
                pltpu.make_async_copy(dst_hbm.at[pl.ds(e_lo + b, CH)], dstb[t],
                                      lsem[t]).wait()
                pltpu.make_async_copy(rows_hbm.at[pl.ds(b, CH)], rows[t],
                                      lsem[t]).wait()

            def wait_scat(t):
                pltpu.make_async_copy(rows[t], acc.at[dstb[t]],
                                      ssem[t]).wait()

            def scale(t):
                def grpw(g, _):
                    sl = pl.ds(g * 16, 16)
                    e16 = g * 16 + lanes
                    pv = plsc.load_gather(rows[t], [e16, jnp.zeros((16,), I32)
                                                    + pcol])
                    tv = plsc.load_gather(qtab, [dstb[t][sl]]) + pv
                    wb[sl] = jnp.exp(jnp.maximum(tv, 0.01 * tv))
                    return 0

                lax.fori_loop(0, CH // 16, grpw, 0)

                def edge(e, _):
                    wsp = plsc.load_gather(wb, [jnp.zeros((16,), I32) + e])
                    for kk in range(ncols16):
                        cs = pl.ds(kk * 16, 16)
                        rows[t][e, cs] = rows[t][e, cs] * wsp
                    return 0

                lax.fori_loop(0, CH, edge, 0, unroll=4)

            def slot(j, t0):
                t1 = (t0 + 1) % NB
                j1 = j + 1

                @pl.when(j1 < NCH)
                def _():
                    @pl.when(j1 >= NB)
                    def _():
                        wait_scat(t1)

                    issue_load(j1, t1)

                wait_load(j, t0)
                scale(t0)
                pltpu.async_copy(rows[t0], acc.at[dstb[t0]], ssem[t0],
                                 add=True)

            issue_load(0, 0)

            def big(m, _):
                for t in range(NB):
                    j = m * NB + t

                    @pl.when(j < NCH)
                    def _():
                        slot(j, t)

                return 0

            lax.fori_loop(0, (NCH + NB - 1) // NB, big, 0)
            for j in range(NCH - NB, NCH):
                wait_scat(j % NB)

        @pl.when(c == 0)
        def _():
            run(lo_hbm, H // 16, LO)

        @pl.when(c == 1)
        def _():
            run(hi_hbm, (NCOL_HI + 15) // 16, PCOL_HI)

        plsc.subcore_barrier()
        r0 = s * NPT

        @pl.when(c == 0)
        def _():
            pltpu.sync_copy(acc.at[pl.ds(r0, NPT)], olo.at[pl.ds(r0, NPT)])

        @pl.when(c == 1)
        def _():
            pltpu.sync_copy(acc.at[pl.ds(r0, NPT)], ohi.at[pl.ds(r0, NPT)])

    return k(lo, hi, q1, dst, zrows)


def _sc_scatter2(pn_lo, pn_hi, qd, src, dst, zrows):
    mesh = plsc.VectorSubcoreMesh(core_axis_name="c", subcore_axis_name="s")
    EPT = E // 16
    CH = 80
    NCH = EPT // CH
    NPT = NP // 16

    @functools.partial(
        pl.kernel,
        out_type=(jax.ShapeDtypeStruct((NP, H), F32),
                  jax.ShapeDtypeStruct((NP, H), F32)),
        mesh=mesh,
        compiler_params=pltpu.CompilerParams(needs_layout_passes=False),
        scratch_types=[
            pltpu.VMEM((NP,), F32),
            [pltpu.VMEM((CH,), I32)] * 3,
            [pltpu.VMEM((CH,), I32)] * 3,
            pltpu.VMEM((CH,), F32),
            [pltpu.VMEM((CH, H), F32)] * 2,
            pltpu.VMEM_SHARED((NP, H), F32),
            [pltpu.SemaphoreType.DMA] * 3,
            [pltpu.SemaphoreType.DMA] * 2,
            [pltpu.SemaphoreType.DMA] * 2,
        ],
    )
    def k(lo_hbm, hi_hbm, qd_hbm, src_hbm, dst_hbm, z_hbm, olo, ohi,
          qdtab, dstb, srcb, wb, rows, acc, lsem, gsem, ssem):
        c = lax.axis_index("c")
        s = lax.axis_index("s")
        pltpu.sync_copy(qd_hbm, qdtab)
        pltpu.sync_copy(z_hbm, acc.at[pl.ds(s * NPT, NPT)])
        plsc.subcore_barrier()

        def run(rows_hbm, ncols16, pcol):
            e0 = s * EPT
            lanes = lax.iota(I32, 16)

            def issue_idx(j, t3):
                b = e0 + j * CH
                pltpu.async_copy(dst_hbm.at[pl.ds(b, CH)], dstb[t3], lsem[t3])
                pltpu.async_copy(src_hbm.at[pl.ds(b, CH)], srcb[t3], lsem[t3])

            def wait_idx(j, t3):
                b = e0 + j * CH
                pltpu.make_async_copy(dst_hbm.at[pl.ds(b, CH)], dstb[t3],
                                      lsem[t3]).wait()
                pltpu.make_async_copy(src_hbm.at[pl.ds(b, CH)], srcb[t3],
                                      lsem[t3]).wait()

            def wait_scat(t2, t3):
                pltpu.make_async_copy(rows[t2], acc.at[dstb[t3]],
                                      ssem[t2]).wait()

            def scale(t2, t3):
                def grpw(g, _):
                    sl = pl.ds(g * 16, 16)
                    e16 = g * 16 + lanes
                    qsv = plsc.load_gather(rows[t2], [e16, jnp.zeros((16,), I32)
                                                      + pcol])
                    tv = plsc.load_gather(qdtab, [dstb[t3][sl]]) + qsv
                    wb[sl] = jnp.exp(jnp.maximum(tv, 0.01 * tv))
                    return 0

                lax.fori_loop(0, CH // 16, grpw, 0)

                def edge(e, _):
                    wsp = plsc.load_gather(wb, [jnp.zeros((16,), I32) + e])
                    for kk in range(ncols16):
                        cs = pl.ds(kk * 16, 16)
                        rows[t2][e, cs] = rows[t2][e, cs] * wsp
                    return 0

                lax.fori_loop(0, CH, edge, 0, unroll=4)

            def slot(j, t):
                t2 = t % 2
                t3 = t % 3
                j1 = j + 1
                j2 = j + 2

                def drain():
                    wait_scat((t - 1) % 2, (t - 1) % 3)

                if t == 0:
                    @pl.when(j >= 1)
                    def _():
                        drain()
                else:
                    drain()

                @pl.when(j2 < NCH)
                def _():
                    issue_idx(j2, (t + 2) % 3)

                @pl.when(j1 < NCH)
                def _():
                    wait_idx(j1, (t + 1) % 3)
                    pltpu.async_copy(rows_hbm.at[srcb[(t + 1) % 3]],
                                     rows[(t + 1) % 2], gsem[(t + 1) % 2])

                pltpu.make_async_copy(rows_hbm.at[srcb[t3]], rows[t2],
                                      gsem[t2]).wait()
                scale(t2, t3)
                pltpu.async_copy(rows[t2], acc.at[dstb[t3]], ssem[t2],
                                 add=True)

            issue_idx(0, 0)
            issue_idx(1, 1)
            wait_idx(0, 0)
            pltpu.async_copy(rows_hbm.at[srcb[0]], rows[0], gsem[0])

            def big(m, _):
                for t in range(6):
                    j = m * 6 + t

                    @pl.when(j < NCH)
                    def _():
                        slot(j, t)

                return 0

            lax.fori_loop(0, (NCH + 5) // 6, big, 0)
            wait_scat((NCH - 1) % 2, (NCH - 1) % 3)

        @pl.when(c == 0)
        def _():
            run(lo_hbm, H // 16, LO)

        @pl.when(c == 1)
        def _():
            run(hi_hbm, (NCOL_HI + 15) // 16, PCOL_HI)

        plsc.subcore_barrier()
        r0 = s * NPT

        @pl.when(c == 0)
        def _():
            pltpu.sync_copy(acc.at[pl.ds(r0, NPT)], olo.at[pl.ds(r0, NPT)])

        @pl.when(c == 1)
        def _():
            pltpu.sync_copy(acc.at[pl.ds(r0, NPT)], ohi.at[pl.ds(r0, NPT)])

    return k(pn_lo, pn_hi, qd, src, dst, zrows)


def _tc_node1(xpad, Wpn, bpn, Wq, bq):
    BN = 1264

    def body(x_ref, wpn, bpn_r, wq, bq_r, hv_ref, q_ref):
        hv = _lrelu(_dot(x_ref[...], wpn[...]) + bpn_r[...])
        hv_ref[...] = hv
        q_ref[...] = _dot(hv, wq[...]) + bq_r[...]

    return pl.pallas_call(
        body,
        grid=(NP // BN,),
        in_specs=[
            pl.BlockSpec((BN, XW), lambda i: (i, 0)),
            pl.BlockSpec((XW, G), lambda i: (0, 0)),
            pl.BlockSpec((1, G), lambda i: (0, 0)),
            pl.BlockSpec((G, 1), lambda i: (0, 0)),
            pl.BlockSpec((1, 1), lambda i: (0, 0)),
        ],
        out_specs=[
            pl.BlockSpec((BN, G), lambda i: (i, 0)),
            pl.BlockSpec((BN, 1), lambda i: (i, 0)),
        ],
        out_shape=[
            jax.ShapeDtypeStruct((NP, G), F32),
            jax.ShapeDtypeStruct((NP, 1), F32),
        ],
    )(xpad, Wpn, bpn, Wq, bq)


def _tc_edge(xg, ea, Wa, Wb, b1, Wp, e_lo, ne):
    BE = 1280

    def body(xg_ref, ea_ref, wa, wb_r, b1_r, wp, lo_ref, hi_ref):
        he = _lrelu(_dot(xg_ref[...], wa[...])
                    + _dot(ea_ref[...], wb_r[...]) + b1_r[...])
        pcol = _dot(he, wp[...])
        lo_ref[...] = jnp.concatenate([he[:, :LO], pcol], axis=1)
        hi_ref[...] = jnp.concatenate(
            [he[:, LO:], jnp.ones((BE, 1), F32), pcol,
             jnp.zeros((BE, H - NCOL_HI), F32)], axis=1)

    blk0 = e_lo // BE
    return pl.pallas_call(
        body,
        grid=(ne // BE,),
        in_specs=[
            pl.BlockSpec((BE, XW), lambda i: (i + blk0, 0)),
            pl.BlockSpec((BE, EF), lambda i: (i + blk0, 0)),
            pl.BlockSpec((XW, G), lambda i: (0, 0)),
            pl.BlockSpec((EF, G), lambda i: (0, 0)),
            pl.BlockSpec((1, G), lambda i: (0, 0)),
            pl.BlockSpec((G, 1), lambda i: (0, 0)),
        ],
        out_specs=[
            pl.BlockSpec((BE, H), lambda i: (i, 0)),
            pl.BlockSpec((BE, H), lambda i: (i, 0)),
        ],
        out_shape=[
            jax.ShapeDtypeStruct((ne, H), F32),
            jax.ShapeDtypeStruct((ne, H), F32),
        ],
    )(xg, ea, Wa, Wb, b1, Wp)


def _tc_node2(acc_a, acc_b, hv, Wet, bet, gru1, Wplt, Wplb, bpl, Wpn2, bpn2):
    BN = 1264

    def body(alo_a, ahi_a, alo_b, ahi_b, hv_ref, wet, bet_r, *rest):
        g1 = rest[:12]
        wplt, wplb, bpl_r, wpn2, bpn2_r = rest[12:17]
        h_o, qd_o, plo_o, phi_o = rest[17:]
        alo = alo_a[...] + alo_b[...]
        ahiv = ahi_a[...] + ahi_b[...]
        sden = ahiv[:, SCOL:SCOL + 1]
        safe = jnp.where(sden == 0.0, 1.0, sden)
        ind = (sden > 0.0).astype(F32)
        accf = jnp.concatenate([alo[:, :LO], ahiv[:, :SCOL]], axis=1) / safe
        ctx = _elu(_dot(accf, wet[...]) + bet_r[...] * ind)
        h = jax.nn.relu(_gru(ctx, hv_ref[...], *g1))
        h_o[...] = h
        qd_o[...] = _dot(h, wplt[...])
        qs = _dot(h, wplb[...]) + bpl_r[...]
        pn = _dot(h, wpn2[...]) + bpn2_r[...]
        plo_o[...] = jnp.concatenate([pn[:, :LO], qs], axis=1)
        phi_o[...] = jnp.concatenate(
            [pn[:, LO:], jnp.ones((BN, 1), F32), qs,
             jnp.zeros((BN, H - NCOL_HI), F32)], axis=1)

    full = lambda shape: pl.BlockSpec(shape, lambda i: tuple(0 for _ in shape))
    return pl.pallas_call(
        body,
        grid=(NP // BN,),
        in_specs=[
            pl.BlockSpec((BN, H), lambda i: (i, 0)),
            pl.BlockSpec((BN, H), lambda i: (i, 0)),
            pl.BlockSpec((BN, H), lambda i: (i, 0)),
            pl.BlockSpec((BN, H), lambda i: (i, 0)),
            pl.BlockSpec((BN, G), lambda i: (i, 0)),
            full((G, G)), full((1, G)),
            *[full(w.shape) for w in gru1],
            full((G, 1)), full((G, 1)), full((1, 1)),
            full((G, G)), full((1, G)),
        ],
        out_specs=[
            pl.BlockSpec((BN, G), lambda i: (i, 0)),
            pl.BlockSpec((BN, 1), lambda i: (i, 0)),
            pl.BlockSpec((BN, H), lambda i: (i, 0)),
            pl.BlockSpec((BN, H), lambda i: (i, 0)),
        ],
        out_shape=[
            jax.ShapeDtypeStruct((NP, G), F32),
            jax.ShapeDtypeStruct((NP, 1), F32),
            jax.ShapeDtypeStruct((NP, H), F32),
            jax.ShapeDtypeStruct((NP, H), F32),
        ],
    )(acc_a[0], acc_a[1], acc_b[0], acc_b[1], hv, Wet, bet, *gru1,
      Wplt, Wplb, bpl, Wpn2, bpn2)


def _tc_readout(acc_lo, acc_hi, h, gid_row, gid_col, plist):
    CK = 632
    NCH = NP // CK
    npar = len(plist)

    def body(*refs):
        alo, ahi, h_ref, gr, gc = refs[:5]
        prefs = refs[5:5 + npar]
        out_ref = refs[5 + npar]
        h2s = refs[5 + npar + 1]
        g2w = prefs[0:12]
        i256c = lax.broadcasted_iota(I32, (NG, 1), 0)
        i256r = lax.broadcasted_iota(I32, (1, NG), 1)

        def passA(i, g0):
            slr = pl.ds(i * CK, CK)
            ahic = ahi[slr, :]
            s2 = ahic[:, SCOL:SCOL + 1]
            safe2 = jnp.where(s2 == 0.0, 1.0, s2)
            accf = jnp.concatenate([alo[slr, :LO], ahic[:, :SCOL]],
                                   axis=1) / safe2
            ctx2 = _elu(accf)
            h2c = jax.nn.relu(_gru(ctx2, h_ref[slr, :], *g2w))
            h2s[slr, :] = h2c
            S = (i256c == gr[pl.ds(i, 1), :]).astype(F32)
            return g0 + _dot(S, h2c)

        g = lax.fori_loop(0, NCH, passA, jnp.zeros((NG, G), F32))

        idx = 12
        for _t in range(2):
            wclt, wclb, bcl, wprn, bprn = prefs[idx:idx + 5]
            grt = prefs[idx + 5:idx + 17]
            idx += 17
            qg = _dot(jax.nn.relu(g), wclt[...])

            def passB(i, carry):
                acc, ss = carry
                slr = pl.ds(i * CK, CK)
                h2c = h2s[slr, :]
                qh = _dot(h2c, wclb[...]) + bcl[...]
                St = (gc[slr, :] == i256r).astype(F32)
                tt = _dot(St, qg) + qh
                w = jnp.exp(jnp.maximum(tt, 0.01 * tt))
                S = (i256c == gr[pl.ds(i, 1), :]).astype(F32)
                return acc + _dot(S, h2c * w), ss + _dot(S, w)

            acc, ss = lax.fori_loop(
                0, NCH, passB,
                (jnp.zeros((NG, G), F32), jnp.zeros((NG, 1), F32)))
            safe = jnp.where(ss == 0.0, 1.0, ss)
            ind = (ss > 0.0).astype(F32)
            ctx = _elu(_dot(acc / safe, wprn[...]) + bprn[...] * ind)
            g = jax.nn.relu(_gru(ctx, g, *grt))

        out_ref[...] = _dot(g, prefs[-2][...]) + prefs[-1][...]

    vspec = pl.BlockSpec(memory_space=pltpu.VMEM)
    return pl.pallas_call(
        body,
        in_specs=[vspec] * (5 + npar),
        out_specs=vspec,
        out_shape=jax.ShapeDtypeStruct((NG, 1), F32),
        scratch_shapes=[pltpu.VMEM((NP, G), F32)],
    )(acc_lo, acc_hi, h, gid_row, gid_col, *plist)


def kernel(x, edge_attr, edge_index, node_graph_ids, params):
    p = params
    src = edge_index[0]
    dst = edge_index[1]

    xpad = jnp.pad(x, ((0, NP - N), (0, XW - NF)))
    Wa = jnp.pad(p['W_pe1'][:NF], ((0, XW - NF), (0, 0)))
    Wb = p['W_pe1'][NF:]
    Wpn = jnp.pad(p['W_pn'], ((0, XW - NF), (0, 0)))
    bpn = p['b_pn'][None, :]
    Wq1 = p['W_pe2'][:G]
    bq1 = p['b_pe2'][None, :]
    Wp1 = p['W_pe2'][G:]
    b_pe1 = p['b_pe1'][None, :]
    gru1 = _gru_split(p['Wih1'], p['Whh1'], p['bih1'], p['bhh1'])
    gru2 = _gru_split(p['Wih2'], p['Whh2'], p['bih2'], p['bhh2'])
    Wplt = p['W_pe_l'][:G]
    Wplb = p['W_pe_l'][G:]
    bpl = p['b_pe_l'][None, :]
    zrows = jnp.zeros((NP // 16, H), F32)
    gid_pad = jnp.pad(node_graph_ids, (0, NP - N), constant_values=NG + 7)
    gid_row = gid_pad.reshape(NP // 632, 632)
    gid_col = gid_pad[:, None]

    plist = list(gru2)
    for t in range(2):
        plist += [p['Wcl%d' % t][:G], p['Wcl%d' % t][G:], p['bcl%d' % t][None, :],
                  p['Wprn%d' % t], p['bprn%d' % t][None, :]]
        plist += list(_gru_split(p['Wih_r%d' % t], p['Whh_r%d' % t],
                                 p['bih_r%d' % t], p['bhh_r%d' % t]))
    plist += [p['W_out'], p['b_out'][None, :]]

    EA = 101120
    xg = _sc_gather_x(xpad, src)
    hv, q1c = _tc_node1(xpad, Wpn, bpn, Wq1, bq1)
    q1 = q1c.reshape(NP)
    lo_a, hi_a = _tc_edge(xg, edge_attr, Wa, Wb, b_pe1, Wp1, 0, EA)
    acc_a = _sc_scatter1(lo_a, hi_a, q1, dst, zrows, 0, EA)
    lo_b, hi_b = _tc_edge(xg, edge_attr, Wa, Wb, b_pe1, Wp1, EA, E - EA)
    acc_b = _sc_scatter1(lo_b, hi_b, q1, dst, zrows, EA, E - EA)
    h, qd_c, pn_lo, pn_hi = _tc_node2(acc_a, acc_b, hv, p['W_et'],
                                      p['b_et'][None, :], gru1,
                                      Wplt, Wplb, bpl,
                                      p['W_pn2'], p['b_pn2'][None, :])
    acc2_lo, acc2_hi = _sc_scatter2(pn_lo, pn_hi, qd_c.reshape(NP), src, dst,
                                    zrows)
    return _tc_readout(acc2_lo, acc2_hi, h, gid_row, gid_col, plist)
